# initial kernel scaffold (unmeasured)
import jax
import jax.numpy as jnp
from jax import lax
from jax.experimental import pallas as pl
from jax.experimental.pallas import tpu as pltpu

S = 1024
H = 16
D = 128
SCALE = D ** -0.5


def kernel(Q, K, V):
    def body(q_ref, k_ref, v_ref, out_ref, kr_ref, vr_ref, send_sems, recv_sems):
        my_x = lax.axis_index("x")
        my_y = lax.axis_index("y")
        nbr = (1 - my_x, my_y)

        barrier_sem = pltpu.get_barrier_semaphore()
        pl.semaphore_signal(
            barrier_sem, inc=1, device_id=nbr,
            device_id_type=pl.DeviceIdType.MESH,
        )
        pl.semaphore_wait(barrier_sem, 1)

        rdma_k = pltpu.make_async_remote_copy(
            src_ref=k_ref, dst_ref=kr_ref,
            send_sem=send_sems.at[0], recv_sem=recv_sems.at[0],
            device_id=nbr, device_id_type=pl.DeviceIdType.MESH,
        )
        rdma_v = pltpu.make_async_remote_copy(
            src_ref=v_ref, dst_ref=vr_ref,
            send_sem=send_sems.at[1], recv_sem=recv_sems.at[1],
            device_id=nbr, device_id_type=pl.DeviceIdType.MESH,
        )
        rdma_k.start()
        rdma_v.start()
        rdma_k.wait()
        rdma_v.wait()

        def qk(a, b):
            return lax.dot_general(
                a, b, (((1,), (1,)), ((), ())),
                preferred_element_type=jnp.float32,
            )

        def pv(p, v):
            return lax.dot_general(
                p, v, (((1,), (0,)), ((), ())),
                preferred_element_type=jnp.float32,
            )

        for h in range(H):
            qh = q_ref[0, :, h, :]
            s1 = qk(qh, k_ref[0, :, h, :]) * SCALE
            s2 = qk(qh, kr_ref[0, :, h, :]) * SCALE
            m = jnp.maximum(
                jnp.max(s1, axis=1, keepdims=True),
                jnp.max(s2, axis=1, keepdims=True),
            )
            p1 = jnp.exp(s1 - m)
            p2 = jnp.exp(s2 - m)
            l = jnp.sum(p1, axis=1, keepdims=True) + jnp.sum(p2, axis=1, keepdims=True)
            o = (pv(p1, v_ref[0, :, h, :]) + pv(p2, vr_ref[0, :, h, :])) / l
            out_ref[0, :, h, :] = o

    return pl.pallas_call(
        body,
        out_shape=jax.ShapeDtypeStruct((1, S, H, D), jnp.float32),
        in_specs=[pl.BlockSpec(memory_space=pltpu.VMEM)] * 3,
        out_specs=pl.BlockSpec(memory_space=pltpu.VMEM),
        scratch_shapes=[
            pltpu.VMEM((1, S, H, D), jnp.float32),
            pltpu.VMEM((1, S, H, D), jnp.float32),
            pltpu.SemaphoreType.DMA((2,)),
            pltpu.SemaphoreType.DMA((2,)),
        ],
        compiler_params=pltpu.CompilerParams(collective_id=0),
    )(Q, K, V)


# baseline (device time: 216973 ns/iter reference)
import jax
import jax.numpy as jnp
from jax import lax
from jax.experimental import pallas as pl
from jax.experimental.pallas import tpu as pltpu

S = 1024
H = 16
D = 128
SCALE = D ** -0.5


def kernel(Q, K, V):
    def body(q_ref, k_ref, v_ref, k_any, v_any, out_ref,
             kr_ref, vr_ref, send_sems, recv_sems):
        h = pl.program_id(0)
        my_x = lax.axis_index("x")
        my_y = lax.axis_index("y")
        nbr = (1 - my_x, my_y)

        def chunk_rdma(tensor_idx, any_ref, remote_ref, hh):
            return pltpu.make_async_remote_copy(
                src_ref=any_ref.at[:, pl.ds(hh * D, D)],
                dst_ref=remote_ref.at[hh],
                send_sem=send_sems.at[tensor_idx, hh],
                recv_sem=recv_sems.at[tensor_idx, hh],
                device_id=nbr, device_id_type=pl.DeviceIdType.MESH,
            )

        @pl.when(h == 0)
        def _comm():
            barrier_sem = pltpu.get_barrier_semaphore()
            pl.semaphore_signal(
                barrier_sem, inc=1, device_id=nbr,
                device_id_type=pl.DeviceIdType.MESH,
            )
            pl.semaphore_wait(barrier_sem, 1)
            for hh in range(H):
                chunk_rdma(0, k_any, kr_ref, hh).start()
                chunk_rdma(1, v_any, vr_ref, hh).start()

        recv_k = pltpu.make_async_remote_copy(
            src_ref=kr_ref.at[h], dst_ref=kr_ref.at[h],
            send_sem=send_sems.at[0, 0], recv_sem=recv_sems.at[0, h],
            device_id=nbr, device_id_type=pl.DeviceIdType.MESH,
        )
        recv_v = pltpu.make_async_remote_copy(
            src_ref=vr_ref.at[h], dst_ref=vr_ref.at[h],
            send_sem=send_sems.at[1, 0], recv_sem=recv_sems.at[1, h],
            device_id=nbr, device_id_type=pl.DeviceIdType.MESH,
        )
        recv_k.wait_recv()
        recv_v.wait_recv()

        def qk(a, b):
            return lax.dot_general(
                a, b, (((1,), (1,)), ((), ())),
                preferred_element_type=jnp.float32,
            )

        def pv(p, v):
            return lax.dot_general(
                p, v, (((1,), (0,)), ((), ())),
                preferred_element_type=jnp.float32,
            )

        qh = q_ref[:, :]
        s1 = qk(qh, k_ref[:, :]) * SCALE
        s2 = qk(qh, kr_ref[h]) * SCALE
        m = jnp.maximum(
            jnp.max(s1, axis=1, keepdims=True),
            jnp.max(s2, axis=1, keepdims=True),
        )
        p1 = jnp.exp(s1 - m)
        p2 = jnp.exp(s2 - m)
        l = jnp.sum(p1, axis=1, keepdims=True) + jnp.sum(p2, axis=1, keepdims=True)
        o = (pv(p1, v_ref[:, :]) + pv(p2, vr_ref[h])) / l
        out_ref[:, :] = o

        @pl.when(h == H - 1)
        def _drain():
            for hh in range(H):
                chunk_rdma(0, k_any, kr_ref, hh).wait_send()
                chunk_rdma(1, v_any, vr_ref, hh).wait_send()

    head_block = pl.BlockSpec((S, D), lambda h: (0, h))
    Q2 = Q.reshape(S, H * D)
    K2 = K.reshape(S, H * D)
    V2 = V.reshape(S, H * D)
    out2 = pl.pallas_call(
        body,
        grid=(H,),
        out_shape=jax.ShapeDtypeStruct((S, H * D), jnp.float32),
        in_specs=[
            head_block,
            head_block,
            head_block,
            pl.BlockSpec(memory_space=pl.ANY),
            pl.BlockSpec(memory_space=pl.ANY),
        ],
        out_specs=head_block,
        scratch_shapes=[
            pltpu.VMEM((H, S, D), jnp.float32),
            pltpu.VMEM((H, S, D), jnp.float32),
            pltpu.SemaphoreType.DMA((2, H)),
            pltpu.SemaphoreType.DMA((2, H)),
        ],
        compiler_params=pltpu.CompilerParams(collective_id=0),
    )(Q2, K2, V2, K2, V2)
    return out2.reshape(1, S, H, D)
